# baseline (speedup 1.0000x reference)
import functools

import jax
import jax.numpy as jnp
from jax import lax
from jax.experimental import pallas as pl
from jax.experimental.pallas import tpu as pltpu
from jax.experimental.pallas import tpu_sc as plsc

_MARGIN = 0.5
_NON_MATCH_LOSS_WEIGHT = 1.0
_L = 16


def _sc_geometry():
    try:
        info = plsc.get_sparse_core_info()
        return info.num_cores, info.num_subcores
    except Exception:
        return 2, 16


@functools.partial(jax.jit, static_argnums=(6, 7, 8, 9))
def _partials(ta, tb, mA, mB, nA, nB, B, N, D, M):
    NC, NS = _sc_geometry()
    NW = NC * NS
    WPB = NW // B
    DPW = D // WPB
    M2 = M // 2
    mesh = plsc.VectorSubcoreMesh(core_axis_name="c", subcore_axis_name="s",
                                  num_cores=NC, num_subcores=NS)

    def body(ta_hbm, tb_hbm, mA_hbm, mB_hbm, nA_hbm, nB_hbm, out_hbm,
             rowa, rowb, ima, imb, ina, inb, res_v, semr, semi):
        wid = lax.axis_index("s") * NC + lax.axis_index("c")
        b = wid // WPB
        dg = (wid % WPB) * DPW

        cps = [pltpu.async_copy(src.at[pl.ds(b * M2, M2)], dst, semi)
               for src, dst in ((mA_hbm, ima), (mB_hbm, imb),
                                (nA_hbm, ina), (nB_hbm, inb))]
        for cp in cps:
            cp.wait()

        acc_m = [jnp.zeros((_L,), jnp.float32) for _ in range(4)]
        acc_p = [jnp.zeros((_L,), jnp.float32) for _ in range(4)]
        acc_c = [jnp.zeros((_L,), jnp.float32) for _ in range(4)]

        for dd in range(DPW):
            d1 = dg + dd
            ca = pltpu.async_copy(ta_hbm.at[b, d1, :], rowa, semr)
            cb = pltpu.async_copy(tb_hbm.at[b, d1, :], rowb, semr)
            ca.wait()
            cb.wait()
            def mbody(i, accs):
                accs = list(accs)
                sl = pl.ds(i * _L, _L)
                wa = ima[sl]
                wb = imb[sl]
                for half in range(2):
                    if half == 0:
                        ia = wa & 0xFFFF
                        ib = wb & 0xFFFF
                    else:
                        ia = lax.shift_right_logical(wa, 16)
                        ib = lax.shift_right_logical(wb, 16)
                    av = plsc.load_gather(rowa, [ia])
                    bv = plsc.load_gather(rowb, [ib])
                    d_ = av - bv
                    accs[2 * half] = accs[2 * half] + d_ * d_
                return tuple(accs)
            acc_m = list(lax.fori_loop(0, M2 // _L, mbody, tuple(acc_m)))

            def nbody(i, accs):
                a0 = list(accs[0])
                a1 = list(accs[1])
                sl = pl.ds(i * _L, _L)
                wa = ina[sl]
                wb = inb[sl]
                for half in range(2):
                    if half == 0:
                        ia = wa & 0xFFFF
                        ib = wb & 0xFFFF
                    else:
                        ia = lax.shift_right_logical(wa, 16)
                        ib = lax.shift_right_logical(wb, 16)
                    av = plsc.load_gather(rowa, [ia])
                    bv = plsc.load_gather(rowb, [ib])
                    d_ = av - bv
                    t = _MARGIN - d_ * d_
                    pos = t > 0.0
                    a0[2 * half] = a0[2 * half] + jnp.where(pos, t, 0.0)
                    a1[2 * half] = a1[2 * half] + jnp.where(pos, 1.0, 0.0)
                return tuple(a0), tuple(a1)
            acc_p, acc_c = lax.fori_loop(0, M2 // _L, nbody,
                                         (tuple(acc_p), tuple(acc_c)))
            acc_p = list(acc_p)
            acc_c = list(acc_c)

        zero = jnp.zeros((_L,), jnp.float32)
        for j in range(4 * B):
            res_v[pl.ds(j * _L, _L)] = zero
        res_v[pl.ds(b * 64, _L)] = (acc_m[0] + acc_m[1]) + (acc_m[2] + acc_m[3])
        res_v[pl.ds(b * 64 + _L, _L)] = (acc_p[0] + acc_p[1]) + (acc_p[2] + acc_p[3])
        res_v[pl.ds(b * 64 + 2 * _L, _L)] = (acc_c[0] + acc_c[1]) + (acc_c[2] + acc_c[3])
        pltpu.sync_copy(res_v, out_hbm.at[pl.ds(wid * 4 * B * _L, 4 * B * _L)])

    call = pl.kernel(
        body,
        out_type=jax.ShapeDtypeStruct((NW * B * 4 * _L,), jnp.float32),
        mesh=mesh,
        scratch_types=[
            pltpu.VMEM((N,), jnp.float32),
            pltpu.VMEM((N,), jnp.float32),
            pltpu.VMEM((M2,), jnp.int32),
            pltpu.VMEM((M2,), jnp.int32),
            pltpu.VMEM((M2,), jnp.int32),
            pltpu.VMEM((M2,), jnp.int32),
            pltpu.VMEM((B * 4 * _L,), jnp.float32),
            pltpu.SemaphoreType.DMA,
            pltpu.SemaphoreType.DMA,
        ],
        compiler_params=pltpu.CompilerParams(needs_layout_passes=False),
    )
    return call(ta, tb, mA, mB, nA, nB)


def kernel(outA, outB, matchA, matchB, nonMatchA, nonMatchB):
    B, N, D = outA.shape
    M = matchA.shape[1]
    ta = jnp.transpose(outA, (0, 2, 1))
    tb = jnp.transpose(outB, (0, 2, 1))

    def pack(x):
        u = x.astype(jnp.uint16).reshape(B, M // 2, 2)
        return jax.lax.bitcast_convert_type(u, jnp.int32).reshape(-1)
    mA = pack(matchA)
    mB = pack(matchB)
    nA = pack(nonMatchA)
    nB = pack(nonMatchB)
    parts = _partials(ta, tb, mA, mB, nA, nB, B, N, D, M)
    NC, NS = _sc_geometry()
    sums = jnp.sum(parts.reshape(NC * NS, B, 4, _L), axis=(0, 3))
    match_loss = jnp.sum(sums[:, 0]) / M
    non_match_loss = _NON_MATCH_LOSS_WEIGHT * jnp.sum(sums[:, 1] / sums[:, 2])
    return (match_loss + non_match_loss, match_loss, non_match_loss)


# speedup vs baseline: 1.2234x; 1.2234x over previous
import functools

import jax
import jax.numpy as jnp
from jax import lax
from jax.experimental import pallas as pl
from jax.experimental.pallas import tpu as pltpu
from jax.experimental.pallas import tpu_sc as plsc

_MARGIN = 0.5
_NON_MATCH_LOSS_WEIGHT = 1.0
_L = 16


def _sc_geometry():
    try:
        info = plsc.get_sparse_core_info()
        return info.num_cores, info.num_subcores
    except Exception:
        return 2, 16


@functools.partial(jax.jit, static_argnums=(6, 7, 8, 9))
def _partials(ta, tb, mA, mB, nA, nB, B, N, D, M):
    NC, NS = _sc_geometry()
    NW = NC * NS
    WPB = NW // B
    DPW = D // WPB
    M2 = M // 2
    mesh = plsc.VectorSubcoreMesh(core_axis_name="c", subcore_axis_name="s",
                                  num_cores=NC, num_subcores=NS)

    def body(ta_hbm, tb_hbm, mA_hbm, mB_hbm, nA_hbm, nB_hbm, out_hbm,
             rowa, rowb, ima, imb, ina, inb, res_v, semr, semi):
        wid = lax.axis_index("s") * NC + lax.axis_index("c")
        b = wid // WPB
        dg = (wid % WPB) * DPW

        cps = [pltpu.async_copy(src.at[pl.ds(b * M2, M2)], dst, semi)
               for src, dst in ((mA_hbm, ima), (mB_hbm, imb),
                                (nA_hbm, ina), (nB_hbm, inb))]
        for cp in cps:
            cp.wait()

        acc_m = [jnp.zeros((_L,), jnp.float32) for _ in range(4)]
        acc_p = [jnp.zeros((_L,), jnp.float32) for _ in range(4)]
        acc_c = [jnp.zeros((_L,), jnp.float32) for _ in range(4)]

        for dd in range(DPW):
            d1 = dg + dd
            ca = pltpu.async_copy(ta_hbm.at[b, d1, :], rowa, semr)
            cb = pltpu.async_copy(tb_hbm.at[b, d1, :], rowb, semr)
            ca.wait()
            cb.wait()
            def mbody(i, accs):
                accs = list(accs)
                sl = pl.ds(i * _L, _L)
                wa = ima[sl]
                wb = imb[sl]
                for half in range(2):
                    if half == 0:
                        ia = wa & 0xFFFF
                        ib = wb & 0xFFFF
                    else:
                        ia = lax.shift_right_logical(wa, 16)
                        ib = lax.shift_right_logical(wb, 16)
                    av = plsc.load_gather(rowa, [ia])
                    bv = plsc.load_gather(rowb, [ib])
                    d_ = av - bv
                    accs[2 * half] = accs[2 * half] + d_ * d_
                return tuple(accs)
            acc_m = list(lax.fori_loop(0, M2 // _L, mbody, tuple(acc_m)))

            def nbody(i, accs):
                a0 = list(accs[0])
                a1 = list(accs[1])
                sl = pl.ds(i * _L, _L)
                wa = ina[sl]
                wb = inb[sl]
                for half in range(2):
                    if half == 0:
                        ia = wa & 0xFFFF
                        ib = wb & 0xFFFF
                    else:
                        ia = lax.shift_right_logical(wa, 16)
                        ib = lax.shift_right_logical(wb, 16)
                    av = plsc.load_gather(rowa, [ia])
                    bv = plsc.load_gather(rowb, [ib])
                    d_ = av - bv
                    t = _MARGIN - d_ * d_
                    pos = t > 0.0
                    a0[2 * half] = a0[2 * half] + jnp.where(pos, t, 0.0)
                    a1[2 * half] = a1[2 * half] + jnp.where(pos, 1.0, 0.0)
                return tuple(a0), tuple(a1)
            acc_p, acc_c = lax.fori_loop(0, M2 // _L, nbody,
                                         (tuple(acc_p), tuple(acc_c)))
            acc_p = list(acc_p)
            acc_c = list(acc_c)

        zero = jnp.zeros((_L,), jnp.float32)
        for j in range(4 * B):
            res_v[pl.ds(j * _L, _L)] = zero
        res_v[pl.ds(b * 64, _L)] = (acc_m[0] + acc_m[1]) + (acc_m[2] + acc_m[3])
        res_v[pl.ds(b * 64 + _L, _L)] = (acc_p[0] + acc_p[1]) + (acc_p[2] + acc_p[3])
        res_v[pl.ds(b * 64 + 2 * _L, _L)] = (acc_c[0] + acc_c[1]) + (acc_c[2] + acc_c[3])
        pltpu.sync_copy(res_v, out_hbm.at[pl.ds(wid * 4 * B * _L, 4 * B * _L)])

    call = pl.kernel(
        body,
        out_type=jax.ShapeDtypeStruct((NW * B * 4 * _L,), jnp.float32),
        mesh=mesh,
        scratch_types=[
            pltpu.VMEM((N,), jnp.float32),
            pltpu.VMEM((N,), jnp.float32),
            pltpu.VMEM((M2,), jnp.int32),
            pltpu.VMEM((M2,), jnp.int32),
            pltpu.VMEM((M2,), jnp.int32),
            pltpu.VMEM((M2,), jnp.int32),
            pltpu.VMEM((B * 4 * _L,), jnp.float32),
            pltpu.SemaphoreType.DMA,
            pltpu.SemaphoreType.DMA,
        ],
        compiler_params=pltpu.CompilerParams(needs_layout_passes=False),
    )
    return call(ta, tb, mA, mB, nA, nB)


def kernel(outA, outB, matchA, matchB, nonMatchA, nonMatchB):
    B, N, D = outA.shape
    M = matchA.shape[1]
    ta = jnp.transpose(outA, (0, 2, 1))
    tb = jnp.transpose(outB, (0, 2, 1))

    w = jnp.array([1, 65536], dtype=jnp.uint32)

    def pack(x):
        u = x.astype(jnp.uint32).reshape(B, M // 2, 2) * w
        return jnp.sum(u, axis=-1, dtype=jnp.uint32).astype(jnp.int32).reshape(-1)
    mA = pack(matchA)
    mB = pack(matchB)
    nA = pack(nonMatchA)
    nB = pack(nonMatchB)
    parts = _partials(ta, tb, mA, mB, nA, nB, B, N, D, M)
    NC, NS = _sc_geometry()
    sums = jnp.sum(parts.reshape(NC * NS, B, 4, _L), axis=(0, 3))
    match_loss = jnp.sum(sums[:, 0]) / M
    non_match_loss = _NON_MATCH_LOSS_WEIGHT * jnp.sum(sums[:, 1] / sums[:, 2])
    return (match_loss + non_match_loss, match_loss, non_match_loss)


# speedup vs baseline: 1.6641x; 1.3602x over previous
import functools

import jax
import jax.numpy as jnp
from jax import lax
from jax.experimental import pallas as pl
from jax.experimental.pallas import tpu as pltpu
from jax.experimental.pallas import tpu_sc as plsc

_MARGIN = 0.5
_NON_MATCH_LOSS_WEIGHT = 1.0
_L = 16


def _sc_geometry():
    try:
        info = plsc.get_sparse_core_info()
        return info.num_cores, info.num_subcores
    except Exception:
        return 2, 16


@functools.partial(jax.jit, static_argnums=(6, 7, 8, 9))
def _partials(ta, tb, mA, mB, nA, nB, B, N, D, M):
    NC, NS = _sc_geometry()
    NW = NC * NS
    WPB = NW // B
    DPW = D // WPB
    M2 = M // 2
    mesh = plsc.VectorSubcoreMesh(core_axis_name="c", subcore_axis_name="s",
                                  num_cores=NC, num_subcores=NS)

    def body(ta_hbm, tb_hbm, mA_hbm, mB_hbm, nA_hbm, nB_hbm, out_hbm,
             rowa, rowb, ima, imb, ina, inb, res_v, semr, semi):
        wid = lax.axis_index("s") * NC + lax.axis_index("c")
        b = wid // WPB
        dg = (wid % WPB) * DPW

        cps = [pltpu.async_copy(src.at[pl.ds(b * M2, M2)], dst, semi)
               for src, dst in ((mA_hbm, ima), (mB_hbm, imb),
                                (nA_hbm, ina), (nB_hbm, inb))]
        for cp in cps:
            cp.wait()

        acc_m = [jnp.zeros((_L,), jnp.float32) for _ in range(4)]
        acc_p = [jnp.zeros((_L,), jnp.float32) for _ in range(4)]
        acc_c = [jnp.zeros((_L,), jnp.float32) for _ in range(4)]

        for dd in range(DPW):
            d1 = dg + dd
            ca = pltpu.async_copy(ta_hbm.at[b, d1, :], rowa, semr)
            cb = pltpu.async_copy(tb_hbm.at[b, d1, :], rowb, semr)
            ca.wait()
            cb.wait()
            def mbody(i, accs):
                accs = list(accs)
                sl = pl.ds(i * _L, _L)
                wa = ima[sl]
                wb = imb[sl]
                for half in range(2):
                    if half == 0:
                        ia = wa & 0xFFFF
                        ib = wb & 0xFFFF
                    else:
                        ia = lax.shift_right_logical(wa, 16)
                        ib = lax.shift_right_logical(wb, 16)
                    av = plsc.load_gather(rowa, [ia])
                    bv = plsc.load_gather(rowb, [ib])
                    d_ = av - bv
                    accs[2 * half] = accs[2 * half] + d_ * d_
                return tuple(accs)
            acc_m = list(lax.fori_loop(0, M2 // _L, mbody, tuple(acc_m)))

            def nbody(i, accs):
                a0 = list(accs[0])
                a1 = list(accs[1])
                sl = pl.ds(i * _L, _L)
                wa = ina[sl]
                wb = inb[sl]
                for half in range(2):
                    if half == 0:
                        ia = wa & 0xFFFF
                        ib = wb & 0xFFFF
                    else:
                        ia = lax.shift_right_logical(wa, 16)
                        ib = lax.shift_right_logical(wb, 16)
                    av = plsc.load_gather(rowa, [ia])
                    bv = plsc.load_gather(rowb, [ib])
                    d_ = av - bv
                    t = _MARGIN - d_ * d_
                    pos = t > 0.0
                    a0[2 * half] = a0[2 * half] + jnp.where(pos, t, 0.0)
                    a1[2 * half] = a1[2 * half] + jnp.where(pos, 1.0, 0.0)
                return tuple(a0), tuple(a1)
            acc_p, acc_c = lax.fori_loop(0, M2 // _L, nbody,
                                         (tuple(acc_p), tuple(acc_c)))
            acc_p = list(acc_p)
            acc_c = list(acc_c)

        zero = jnp.zeros((_L,), jnp.float32)
        for j in range(4 * B):
            res_v[pl.ds(j * _L, _L)] = zero
        res_v[pl.ds(b * 64, _L)] = (acc_m[0] + acc_m[1]) + (acc_m[2] + acc_m[3])
        res_v[pl.ds(b * 64 + _L, _L)] = (acc_p[0] + acc_p[1]) + (acc_p[2] + acc_p[3])
        res_v[pl.ds(b * 64 + 2 * _L, _L)] = (acc_c[0] + acc_c[1]) + (acc_c[2] + acc_c[3])
        pltpu.sync_copy(res_v, out_hbm.at[pl.ds(wid * 4 * B * _L, 4 * B * _L)])

    call = pl.kernel(
        body,
        out_type=jax.ShapeDtypeStruct((NW * B * 4 * _L,), jnp.float32),
        mesh=mesh,
        scratch_types=[
            pltpu.VMEM((N,), jnp.float32),
            pltpu.VMEM((N,), jnp.float32),
            pltpu.VMEM((M2,), jnp.int32),
            pltpu.VMEM((M2,), jnp.int32),
            pltpu.VMEM((M2,), jnp.int32),
            pltpu.VMEM((M2,), jnp.int32),
            pltpu.VMEM((B * 4 * _L,), jnp.float32),
            pltpu.SemaphoreType.DMA,
            pltpu.SemaphoreType.DMA,
        ],
        compiler_params=pltpu.CompilerParams(needs_layout_passes=False),
    )
    return call(ta, tb, mA, mB, nA, nB)


def kernel(outA, outB, matchA, matchB, nonMatchA, nonMatchB):
    B, N, D = outA.shape
    M = matchA.shape[1]
    ta = jnp.transpose(outA, (0, 2, 1))
    tb = jnp.transpose(outB, (0, 2, 1))

    def pack(x):
        u = x.astype(jnp.uint32)
        return (u[:, 0::2] | (u[:, 1::2] << 16)).astype(jnp.int32).reshape(-1)
    mA = pack(matchA)
    mB = pack(matchB)
    nA = pack(nonMatchA)
    nB = pack(nonMatchB)
    parts = _partials(ta, tb, mA, mB, nA, nB, B, N, D, M)
    NC, NS = _sc_geometry()
    sums = jnp.sum(parts.reshape(NC * NS, B, 4, _L), axis=(0, 3))
    match_loss = jnp.sum(sums[:, 0]) / M
    non_match_loss = _NON_MATCH_LOSS_WEIGHT * jnp.sum(sums[:, 1] / sums[:, 2])
    return (match_loss + non_match_loss, match_loss, non_match_loss)
